# Initial kernel scaffold; baseline (speedup 1.0000x reference)
import jax
import jax.numpy as jnp
from jax.experimental import pallas as pl


def kernel(x, w_enc, w_dec, b_enc, b_pre):
    raise NotImplementedError("write your pallas kernel here")


# speedup vs baseline: 15.3796x; 15.3796x over previous
import functools

import jax
import jax.numpy as jnp
from jax.experimental import pallas as pl
from jax.experimental.pallas import tpu as pltpu

D_MODEL = 1024
D_HIDDEN = 4096
TOPK = 128
BLOCK_T = 256

_INT_MIN = -2147483648


def _fused_body(x_ref, w_enc_ref, w_dec_ref, b_enc_ref, b_pre_ref, out_ref):
    xb = x_ref[...]
    mu = jnp.mean(xb, axis=-1, keepdims=True)
    xc = xb - mu
    var = jnp.sum(xc * xc, axis=-1, keepdims=True) / (D_MODEL - 1)
    std = jnp.sqrt(var)
    b_pre = b_pre_ref[...]
    xn = xc / (std + 1e-5) - b_pre

    pre = (
        jnp.dot(xn, w_enc_ref[...], preferred_element_type=jnp.float32)
        + b_enc_ref[...]
    )

    u = pre.view(jnp.int32)
    key = jnp.where(u >= 0, u, u ^ jnp.int32(0x7FFFFFFF))

    def step(i, t_u):
        cand = t_u | (jnp.int32(1) << (jnp.int32(31) - i))
        cand_key = cand ^ jnp.int32(_INT_MIN)
        cnt = jnp.sum(
            (key >= cand_key).astype(jnp.int32), axis=-1, keepdims=True
        )
        return jnp.where(cnt >= TOPK, cand, t_u)

    t_u = jax.lax.fori_loop(
        0, 32, step, jnp.zeros((BLOCK_T, 1), jnp.int32)
    )
    thresh_key = t_u ^ jnp.int32(_INT_MIN)

    latents = jnp.where(key >= thresh_key, jax.nn.relu(pre), 0.0)

    recons = (
        jnp.dot(latents, w_dec_ref[...], preferred_element_type=jnp.float32)
        + b_pre
    )
    out_ref[...] = recons * std + mu


def _fused_call(x2d, w_enc, w_dec, b_enc2d, b_pre2d):
    n_tok = x2d.shape[0]
    grid = (n_tok // BLOCK_T,)
    return pl.pallas_call(
        _fused_body,
        grid=grid,
        in_specs=[
            pl.BlockSpec((BLOCK_T, D_MODEL), lambda i: (i, 0)),
            pl.BlockSpec((D_MODEL, D_HIDDEN), lambda i: (0, 0)),
            pl.BlockSpec((D_HIDDEN, D_MODEL), lambda i: (0, 0)),
            pl.BlockSpec((1, D_HIDDEN), lambda i: (0, 0)),
            pl.BlockSpec((1, D_MODEL), lambda i: (0, 0)),
        ],
        out_specs=pl.BlockSpec((BLOCK_T, D_MODEL), lambda i: (i, 0)),
        out_shape=jax.ShapeDtypeStruct((n_tok, D_MODEL), jnp.float32),
    )(x2d, w_enc, w_dec, b_enc2d, b_pre2d)


def kernel(x, w_enc, w_dec, b_enc, b_pre):
    b, t, d = x.shape
    x2d = x.reshape(b * t, d)
    b_enc2d = b_enc.reshape(1, D_HIDDEN)
    b_pre2d = b_pre.reshape(1, D_MODEL)
    out = _fused_call(x2d, w_enc, w_dec, b_enc2d, b_pre2d)
    return out.reshape(b, t, d)


# speedup vs baseline: 15.4036x; 1.0016x over previous
import functools

import jax
import jax.numpy as jnp
from jax.experimental import pallas as pl
from jax.experimental.pallas import tpu as pltpu

D_MODEL = 1024
D_HIDDEN = 4096
TOPK = 128
BLOCK_T = 256

_INT_MIN = -2147483648


def _fused_body(x_ref, w_enc_ref, w_dec_ref, b_enc_ref, b_pre_ref, out_ref):
    xb = x_ref[...]
    mu = jnp.mean(xb, axis=-1, keepdims=True)
    xc = xb - mu
    var = jnp.sum(xc * xc, axis=-1, keepdims=True) / (D_MODEL - 1)
    std = jnp.sqrt(var)
    b_pre = b_pre_ref[...]
    xn = xc / (std + 1e-5) - b_pre

    pre = (
        jnp.dot(xn, w_enc_ref[...], preferred_element_type=jnp.float32)
        + b_enc_ref[...]
    )

    u = pre.view(jnp.int32)
    key = jnp.where(u >= 0, u, u ^ jnp.int32(0x7FFFFFFF))

    def step(i, t_u):
        cand = t_u | (jnp.int32(1) << (jnp.int32(31) - i))
        cand_key = cand ^ jnp.int32(_INT_MIN)
        cnt = jnp.sum(
            (key >= cand_key).astype(jnp.int32), axis=-1, keepdims=True
        )
        return jnp.where(cnt >= TOPK, cand, t_u)

    t_u = jax.lax.fori_loop(
        0, 32, step, jnp.zeros((BLOCK_T, 1), jnp.int32)
    )
    thresh_key = t_u ^ jnp.int32(_INT_MIN)

    latents = jnp.where(key >= thresh_key, jax.nn.relu(pre), 0.0)

    recons = (
        jnp.dot(
            latents.astype(jnp.bfloat16),
            w_dec_ref[...].astype(jnp.bfloat16),
            preferred_element_type=jnp.float32,
        )
        + b_pre
    )
    out_ref[...] = recons * std + mu


def _fused_call(x2d, w_enc, w_dec, b_enc2d, b_pre2d):
    n_tok = x2d.shape[0]
    grid = (n_tok // BLOCK_T,)
    return pl.pallas_call(
        _fused_body,
        grid=grid,
        in_specs=[
            pl.BlockSpec((BLOCK_T, D_MODEL), lambda i: (i, 0)),
            pl.BlockSpec((D_MODEL, D_HIDDEN), lambda i: (0, 0)),
            pl.BlockSpec((D_HIDDEN, D_MODEL), lambda i: (0, 0)),
            pl.BlockSpec((1, D_HIDDEN), lambda i: (0, 0)),
            pl.BlockSpec((1, D_MODEL), lambda i: (0, 0)),
        ],
        out_specs=pl.BlockSpec((BLOCK_T, D_MODEL), lambda i: (i, 0)),
        out_shape=jax.ShapeDtypeStruct((n_tok, D_MODEL), jnp.float32),
    )(x2d, w_enc, w_dec, b_enc2d, b_pre2d)


def kernel(x, w_enc, w_dec, b_enc, b_pre):
    b, t, d = x.shape
    x2d = x.reshape(b * t, d)
    b_enc2d = b_enc.reshape(1, D_HIDDEN)
    b_pre2d = b_pre.reshape(1, D_MODEL)
    out = _fused_call(x2d, w_enc, w_dec, b_enc2d, b_pre2d)
    return out.reshape(b, t, d)
